# Initial kernel scaffold; baseline (speedup 1.0000x reference)
import jax
import jax.numpy as jnp
from jax.experimental import pallas as pl


def kernel(z_loga_expert):
    raise NotImplementedError("write your pallas kernel here")


# speedup vs baseline: 20.7412x; 20.7412x over previous
import jax
import jax.numpy as jnp
from jax import lax
from jax.experimental import pallas as pl
from jax.experimental.pallas import tpu as pltpu
from jax.experimental.pallas import tpu_sc as plsc

_NL, _NE, _N = 32, 8, 14336
_G = _NL * _NE
_K = _N // 2
_L = 16
_NW = 32
_GPW = _G // _NW
_NV = _N // _L
_NB = 2048
_NBV = _NB // _L


def _scan_hist(hist_v, lo, cb, shift):
    thr = _K - cb
    lanes = lax.iota(jnp.int32, _L)

    def body(i, st):
        run, found, j, cbadd = st
        h = hist_v[pl.ds(i * _L, _L)]
        cum = plsc.cumsum(h)
        m = (cum + run) >= thr
        lane = plsc.all_reduce_ffs(m)
        lane_s = jnp.max(lane)
        found_here = jnp.logical_and(lane_s < _L, jnp.logical_not(found))
        exc_at = jnp.sum(jnp.where(lanes == lane, cum - h, 0))
        j = jnp.where(found_here, i * _L + lane_s, j)
        cbadd = jnp.where(found_here, run + exc_at, cbadd)
        run = run + jnp.max(cum)
        return run, jnp.logical_or(found, found_here), j, cbadd

    init = (jnp.int32(0), jnp.bool_(False), jnp.int32(0), jnp.int32(0))
    _, _, j, cbadd = lax.fori_loop(0, _NBV, body, init)
    return lo + jnp.left_shift(j, shift), cb + cbadd


def _sc_body(x_hbm, out_hbm, data_v, hist_v):
    wid = lax.axis_index("s") * 2 + lax.axis_index("c")
    base = wid * (_GPW * _N)
    pltpu.sync_copy(x_hbm.at[pl.ds(base, _GPW * _N)], data_v)

    ones_i = jnp.ones((_L,), jnp.int32)
    zeros_i = jnp.zeros((_L,), jnp.int32)

    def zero_hist():
        def zbody(i, _):
            hist_v[pl.ds(i * _L, _L)] = zeros_i
            return 0
        lax.fori_loop(0, _NBV, zbody, 0)

    def per_group(g, carry):
        gbase = g * _N

        zero_hist()

        def r1(i, _):
            off = gbase + i * _L
            v = data_v[pl.ds(off, _L)]
            r = jnp.maximum(v, 0.0)
            data_v[pl.ds(off, _L)] = r
            b = plsc.bitcast(r, jnp.int32)
            plsc.addupdate_scatter(hist_v, [lax.shift_right_logical(b, 20)],
                                   ones_i)
            return 0

        lax.fori_loop(0, _NV, r1, 0)
        lo, cb = _scan_hist(hist_v, jnp.int32(0), jnp.int32(0), 20)

        for shift in (9, 0):
            zero_hist()

            def rn(i, _, lo=lo, shift=shift):
                off = gbase + i * _L
                v = data_v[pl.ds(off, _L)]
                b = plsc.bitcast(v, jnp.int32)
                d = b - lo
                idx = lax.shift_right_arithmetic(d, shift)
                m = jnp.logical_and(d >= 0, idx < _NB)
                idx = jnp.clip(idx, 0, _NB - 1)
                plsc.addupdate_scatter(hist_v, [idx], ones_i, mask=m)
                return 0

            lax.fori_loop(0, _NV, rn, 0)
            lo, cb = _scan_hist(hist_v, lo, cb, shift)

        thresh = lo
        need = _K - cb

        def out_pass(i, run):
            off = gbase + i * _L
            v = data_v[pl.ds(off, _L)]
            b = plsc.bitcast(v, jnp.int32)
            is_lt = b < thresh
            is_eq = b == thresh
            pref = plsc.cumsum(jnp.where(is_eq, 1, 0))
            zero_eq = jnp.logical_and(is_eq, (pref + run) <= need)
            z = jnp.logical_or(is_lt, zero_eq)
            data_v[pl.ds(off, _L)] = jnp.where(z, 0.0, v)
            return run + plsc.all_reduce_population_count(is_eq)

        lax.fori_loop(0, _NV, out_pass, zeros_i)
        return carry

    lax.fori_loop(0, _GPW, per_group, 0)
    pltpu.sync_copy(data_v, out_hbm.at[pl.ds(base, _GPW * _N)])


_mesh = plsc.VectorSubcoreMesh(core_axis_name="c", subcore_axis_name="s",
                               num_cores=2, num_subcores=16)

_sc_call = pl.kernel(
    _sc_body,
    out_type=jax.ShapeDtypeStruct((_G * _N,), jnp.float32),
    mesh=_mesh,
    scratch_types=[
        pltpu.VMEM((_GPW * _N,), jnp.float32),
        pltpu.VMEM((_NB,), jnp.int32),
    ],
    compiler_params=pltpu.CompilerParams(needs_layout_passes=False),
)


@jax.jit
def kernel(z_loga_expert):
    flat = z_loga_expert.reshape(_G * _N)
    out = _sc_call(flat)
    return out.reshape(_NL, _NE, _N)


# speedup vs baseline: 52.0123x; 2.5077x over previous
import jax
import jax.numpy as jnp
from jax import lax
from jax.experimental import pallas as pl
from jax.experimental.pallas import tpu as pltpu
from jax.experimental.pallas import tpu_sc as plsc

_NL, _NE, _N = 32, 8, 14336
_G = _NL * _NE
_K = _N // 2
_L = 16
_NW = 32
_GPW = _G // _NW
_NV = _N // _L
_NB = 2048
_NBV = _NB // _L


def _scan_hist(hist_v, lo, cb, shift):
    thr = _K - cb
    lanes = lax.iota(jnp.int32, _L)

    def body(i, st):
        run, found, j, cbadd = st
        h = hist_v[pl.ds(i * _L, _L)]
        cum = plsc.cumsum(h)
        m = (cum + run) >= thr
        lane = plsc.all_reduce_ffs(m)
        lane_s = jnp.max(lane)
        found_here = jnp.logical_and(lane_s < _L, jnp.logical_not(found))
        exc_at = jnp.sum(jnp.where(lanes == lane, cum - h, 0))
        j = jnp.where(found_here, i * _L + lane_s, j)
        cbadd = jnp.where(found_here, run + exc_at, cbadd)
        run = run + jnp.max(cum)
        return run, jnp.logical_or(found, found_here), j, cbadd

    init = (jnp.int32(0), jnp.bool_(False), jnp.int32(0), jnp.int32(0))
    _, _, j, cbadd = lax.fori_loop(0, _NBV, body, init)
    return lo + jnp.left_shift(j, shift), cb + cbadd


def _sc_body(x_hbm, out_hbm, data_v, hist_v):
    wid = lax.axis_index("s") * 2 + lax.axis_index("c")
    base = wid * (_GPW * _N)
    pltpu.sync_copy(x_hbm.at[pl.ds(base, _GPW * _N)], data_v)

    ones_i = jnp.ones((_L,), jnp.int32)
    zeros_i = jnp.zeros((_L,), jnp.int32)

    def zero_hist():
        @plsc.parallel_loop(0, _NBV, unroll=8)
        def _(i):
            hist_v[pl.ds(i * _L, _L)] = zeros_i

    def per_group(g, carry):
        gbase = g * _N

        zero_hist()

        @plsc.parallel_loop(0, _NV, unroll=8)
        def _(i):
            off = gbase + i * _L
            v = data_v[pl.ds(off, _L)]
            r = jnp.maximum(v, 0.0)
            data_v[pl.ds(off, _L)] = r
            b = plsc.bitcast(r, jnp.int32)
            plsc.addupdate_scatter(hist_v, [lax.shift_right_logical(b, 20)],
                                   ones_i)
        lo, cb = _scan_hist(hist_v, jnp.int32(0), jnp.int32(0), 20)

        for shift in (9, 0):
            zero_hist()

            @plsc.parallel_loop(0, _NV, unroll=8)
            def _(i, lo=lo, shift=shift):
                off = gbase + i * _L
                v = data_v[pl.ds(off, _L)]
                b = plsc.bitcast(v, jnp.int32)
                d = b - lo
                idx = lax.shift_right_arithmetic(d, shift)
                m = jnp.logical_and(d >= 0, idx < _NB)
                idx = jnp.clip(idx, 0, _NB - 1)
                plsc.addupdate_scatter(hist_v, [idx], ones_i, mask=m)
            lo, cb = _scan_hist(hist_v, lo, cb, shift)

        thresh = lo
        need = _K - cb

        @plsc.parallel_loop(0, _NV, unroll=8, carry=zeros_i)
        def _(i, run):
            off = gbase + i * _L
            v = data_v[pl.ds(off, _L)]
            b = plsc.bitcast(v, jnp.int32)
            is_lt = b < thresh
            is_eq = b == thresh
            pref = plsc.cumsum(jnp.where(is_eq, 1, 0))
            zero_eq = jnp.logical_and(is_eq, (pref + run) <= need)
            z = jnp.logical_or(is_lt, zero_eq)
            data_v[pl.ds(off, _L)] = jnp.where(z, 0.0, v)
            return run + plsc.all_reduce_population_count(is_eq)

        return carry

    lax.fori_loop(0, _GPW, per_group, 0)
    pltpu.sync_copy(data_v, out_hbm.at[pl.ds(base, _GPW * _N)])


_mesh = plsc.VectorSubcoreMesh(core_axis_name="c", subcore_axis_name="s",
                               num_cores=2, num_subcores=16)

_sc_call = pl.kernel(
    _sc_body,
    out_type=jax.ShapeDtypeStruct((_G * _N,), jnp.float32),
    mesh=_mesh,
    scratch_types=[
        pltpu.VMEM((_GPW * _N,), jnp.float32),
        pltpu.VMEM((_NB,), jnp.int32),
    ],
    compiler_params=pltpu.CompilerParams(needs_layout_passes=False),
)


@jax.jit
def kernel(z_loga_expert):
    flat = z_loga_expert.reshape(_G * _N)
    out = _sc_call(flat)
    return out.reshape(_NL, _NE, _N)


# speedup vs baseline: 52.6921x; 1.0131x over previous
import jax
import jax.numpy as jnp
from jax import lax
from jax.experimental import pallas as pl
from jax.experimental.pallas import tpu as pltpu
from jax.experimental.pallas import tpu_sc as plsc

_NL, _NE, _N = 32, 8, 14336
_G = _NL * _NE
_K = _N // 2
_L = 16
_NW = 32
_GPW = _G // _NW
_NV = _N // _L
_NB = 2048
_NBV = _NB // _L


_NCH = 8
_VPC = _NBV // _NCH


def _scan_hist(hist_v, lo, cb, shift):
    thr = _K - cb
    lanes = lax.iota(jnp.int32, _L)

    tots = []
    for c in range(_NCH):
        acc = hist_v[pl.ds(c * _VPC * _L, _L)]
        for i in range(1, _VPC):
            acc = acc + hist_v[pl.ds((c * _VPC + i) * _L, _L)]
        tots.append(jnp.sum(acc))

    runb = jnp.int32(0)
    sel = jnp.int32(0)
    found = jnp.bool_(False)
    cum_c = jnp.int32(0)
    for c in range(_NCH):
        cum_n = cum_c + tots[c]
        hit = jnp.logical_and(jnp.logical_not(found), cum_n >= thr)
        sel = jnp.where(hit, c, sel)
        runb = jnp.where(hit, cum_c, runb)
        found = jnp.logical_or(found, hit)
        cum_c = cum_n

    cbase = sel * (_VPC * _L)
    run = runb
    found = jnp.bool_(False)
    j = jnp.int32(0)
    cbadd = jnp.int32(0)
    for i in range(_VPC):
        h = hist_v[pl.ds(cbase + i * _L, _L)]
        cum = plsc.cumsum(h)
        m = (cum + run) >= thr
        lane = plsc.all_reduce_ffs(m)
        lane_s = jnp.max(lane)
        found_here = jnp.logical_and(lane_s < _L, jnp.logical_not(found))
        exc_at = jnp.sum(jnp.where(lanes == lane, cum - h, 0))
        j = jnp.where(found_here, cbase + i * _L + lane_s, j)
        cbadd = jnp.where(found_here, run + exc_at, cbadd)
        run = run + jnp.max(cum)
        found = jnp.logical_or(found, found_here)

    return lo + jnp.left_shift(j, shift), cb + cbadd


def _sc_body(x_hbm, out_hbm, data_v, hist_v):
    wid = lax.axis_index("s") * 2 + lax.axis_index("c")
    base = wid * (_GPW * _N)
    pltpu.sync_copy(x_hbm.at[pl.ds(base, _GPW * _N)], data_v)

    ones_i = jnp.ones((_L,), jnp.int32)
    zeros_i = jnp.zeros((_L,), jnp.int32)

    def zero_hist():
        @plsc.parallel_loop(0, _NBV, unroll=8)
        def _(i):
            hist_v[pl.ds(i * _L, _L)] = zeros_i

    def per_group(g, carry):
        gbase = g * _N

        zero_hist()

        @plsc.parallel_loop(0, _NV, unroll=8)
        def _(i):
            off = gbase + i * _L
            v = data_v[pl.ds(off, _L)]
            r = jnp.maximum(v, 0.0)
            data_v[pl.ds(off, _L)] = r
            b = plsc.bitcast(r, jnp.int32)
            plsc.addupdate_scatter(hist_v, [lax.shift_right_logical(b, 20)],
                                   ones_i)
        lo, cb = _scan_hist(hist_v, jnp.int32(0), jnp.int32(0), 20)

        for shift in (9, 0):
            zero_hist()

            @plsc.parallel_loop(0, _NV, unroll=8)
            def _(i, lo=lo, shift=shift):
                off = gbase + i * _L
                v = data_v[pl.ds(off, _L)]
                b = plsc.bitcast(v, jnp.int32)
                d = b - lo
                idx = lax.shift_right_arithmetic(d, shift)
                m = jnp.logical_and(d >= 0, idx < _NB)
                idx = jnp.clip(idx, 0, _NB - 1)
                plsc.addupdate_scatter(hist_v, [idx], ones_i, mask=m)
            lo, cb = _scan_hist(hist_v, lo, cb, shift)

        thresh = lo
        need = _K - cb

        @plsc.parallel_loop(0, _NV, unroll=8, carry=zeros_i)
        def _(i, run):
            off = gbase + i * _L
            v = data_v[pl.ds(off, _L)]
            b = plsc.bitcast(v, jnp.int32)
            is_lt = b < thresh
            is_eq = b == thresh
            pref = plsc.cumsum(jnp.where(is_eq, 1, 0))
            zero_eq = jnp.logical_and(is_eq, (pref + run) <= need)
            z = jnp.logical_or(is_lt, zero_eq)
            data_v[pl.ds(off, _L)] = jnp.where(z, 0.0, v)
            return run + plsc.all_reduce_population_count(is_eq)

        return carry

    lax.fori_loop(0, _GPW, per_group, 0)
    pltpu.sync_copy(data_v, out_hbm.at[pl.ds(base, _GPW * _N)])


_mesh = plsc.VectorSubcoreMesh(core_axis_name="c", subcore_axis_name="s",
                               num_cores=2, num_subcores=16)

_sc_call = pl.kernel(
    _sc_body,
    out_type=jax.ShapeDtypeStruct((_G * _N,), jnp.float32),
    mesh=_mesh,
    scratch_types=[
        pltpu.VMEM((_GPW * _N,), jnp.float32),
        pltpu.VMEM((_NB,), jnp.int32),
    ],
    compiler_params=pltpu.CompilerParams(needs_layout_passes=False),
)


@jax.jit
def kernel(z_loga_expert):
    flat = z_loga_expert.reshape(_G * _N)
    out = _sc_call(flat)
    return out.reshape(_NL, _NE, _N)


# speedup vs baseline: 75.2122x; 1.4274x over previous
import jax
import jax.numpy as jnp
from jax import lax
from jax.experimental import pallas as pl
from jax.experimental.pallas import tpu as pltpu
from jax.experimental.pallas import tpu_sc as plsc

_NL, _NE, _N = 32, 8, 14336
_G = _NL * _NE
_K = _N // 2
_L = 16
_NW = 32
_GPW = _G // _NW
_NV = _N // _L
_NB = 2048
_NBP = 8192
_PH = _NBP // 2


def _scan_hist(hist_v, lo, cb, shift, nch):
    thr = _K - cb
    lanes = lax.iota(jnp.int32, _L)

    tots = []
    for c in range(nch):
        acc = hist_v[pl.ds(c * 256, _L)]
        for i in range(1, 16):
            acc = acc + hist_v[pl.ds(c * 256 + i * _L, _L)]
        tots.append(jnp.sum(acc))

    runb = jnp.int32(0)
    sel = jnp.int32(0)
    found = jnp.bool_(False)
    cum_c = jnp.int32(0)
    for c in range(nch):
        cum_n = cum_c + tots[c]
        hit = jnp.logical_and(jnp.logical_not(found), cum_n >= thr)
        sel = jnp.where(hit, c, sel)
        runb = jnp.where(hit, cum_c, runb)
        found = jnp.logical_or(found, hit)
        cum_c = cum_n

    jbase = sel * 256
    run = runb
    f2 = jnp.bool_(False)
    j = jnp.int32(0)
    cbadd = jnp.int32(0)
    ceq = jnp.int32(0)
    for i in range(16):
        h = hist_v[pl.ds(jbase + i * _L, _L)]
        cum = plsc.cumsum(h)
        m = (cum + run) >= thr
        lane = plsc.all_reduce_ffs(m)
        lane_s = jnp.max(lane)
        found_here = jnp.logical_and(lane_s < _L, jnp.logical_not(f2))
        onehot = lanes == lane
        exc_at = jnp.sum(jnp.where(onehot, cum - h, 0))
        h_at = jnp.sum(jnp.where(onehot, h, 0))
        j = jnp.where(found_here, jbase + i * _L + lane_s, j)
        cbadd = jnp.where(found_here, run + exc_at, cbadd)
        ceq = jnp.where(found_here, h_at, ceq)
        run = run + jnp.max(cum)
        f2 = jnp.logical_or(f2, found_here)

    return lo + jnp.left_shift(j, shift), cb + cbadd, found, ceq


def _sc_body(x_hbm, out_hbm, data_v, hist_v, in_sem, out_sem):
    wid = lax.axis_index("s") * 2 + lax.axis_index("c")
    base = wid * (_GPW * _N)

    def in_copy(g):
        return pltpu.make_async_copy(
            x_hbm.at[pl.ds(base + g * _N, _N)],
            data_v.at[pl.ds(g * _N, _N)],
            in_sem.at[g])

    def out_copy(g):
        return pltpu.make_async_copy(
            data_v.at[pl.ds(g * _N, _N)],
            out_hbm.at[pl.ds(base + g * _N, _N)],
            out_sem.at[g])

    def fire(g, carry):
        in_copy(g).start()
        return carry

    lax.fori_loop(0, _GPW, fire, 0)

    ones_i = jnp.ones((_L,), jnp.int32)
    zeros_i = jnp.zeros((_L,), jnp.int32)

    def zero_hist(nvec):
        @plsc.parallel_loop(0, nvec, unroll=8)
        def _(i):
            hist_v[pl.ds(i * _L, _L)] = zeros_i

    def per_group(g, t_prev):
        gbase = g * _N
        in_copy(g).wait()

        def masked_round(lo, cb, shift):
            zero_hist(_NB // _L)

            @plsc.parallel_loop(0, _NV, unroll=8)
            def _(i):
                v = data_v[pl.ds(gbase + i * _L, _L)]
                b = plsc.bitcast(jnp.maximum(v, 0.0), jnp.int32)
                d = b - lo
                idx = jnp.right_shift(d, shift)
                m = jnp.logical_and(d >= 0, idx < _NB)
                idx = jnp.clip(idx, 0, _NB - 1)
                plsc.addupdate_scatter(hist_v, [idx], ones_i, mask=m)

            return _scan_hist(hist_v, lo, cb, shift, _NB // 256)

        def pred_path():
            lo_p = t_prev - _PH
            zero_hist(_NBP // _L)

            @plsc.parallel_loop(0, _NV, unroll=8, carry=zeros_i)
            def below(i, cnt):
                v = data_v[pl.ds(gbase + i * _L, _L)]
                b = plsc.bitcast(jnp.maximum(v, 0.0), jnp.int32)
                d = b - lo_p
                m = jnp.logical_and(d >= 0, d < _NBP)
                idx = jnp.clip(d, 0, _NBP - 1)
                plsc.addupdate_scatter(hist_v, [idx], ones_i, mask=m)
                return cnt + jnp.where(d < 0, 1, 0)

            h_below = jnp.sum(below)
            t, cb, found, ceq = _scan_hist(hist_v, lo_p, h_below, 0,
                                           _NBP // 256)
            ok = jnp.logical_and(h_below < _K, found)
            return t, cb, ceq, ok

        def std_path():
            inf_v = jnp.full((_L,), jnp.inf, jnp.float32)
            zf_v = jnp.zeros((_L,), jnp.float32)

            @plsc.parallel_loop(0, _NV, unroll=8, carry=(inf_v, zf_v))
            def mm(i, c):
                mn, mx = c
                v = data_v[pl.ds(gbase + i * _L, _L)]
                r = jnp.maximum(v, 0.0)
                return jnp.minimum(mn, r), jnp.maximum(mx, r)

            mn_v, mx_v = mm
            mnb = jnp.min(plsc.bitcast(mn_v, jnp.int32))
            mxb = jnp.max(plsc.bitcast(mx_v, jnp.int32))
            rng1 = mxb - mnb
            s0 = jnp.int32(0)
            for t in range(11, 31):
                s0 = s0 + jnp.where(lax.shift_right_logical(rng1, t) != 0,
                                    1, 0)

            lo, cb, _f, ceq = masked_round(mnb, jnp.int32(0), s0)
            s1 = jnp.maximum(s0 - 11, 0)

            def rb(lo=lo, cb=cb):
                return masked_round(lo, cb, s1)

            lo, cb, _f, ceq = lax.cond(
                s0 > 0, rb, lambda: (lo, cb, jnp.bool_(True), ceq))

            def rc(lo=lo, cb=cb):
                return masked_round(lo, cb, jnp.int32(0))

            lo, cb, _f, ceq = lax.cond(
                s1 > 0, rc, lambda: (lo, cb, jnp.bool_(True), ceq))
            return lo, cb, ceq

        t1, cb1, ceq1, ok = lax.cond(
            g > 0, pred_path,
            lambda: (jnp.int32(0), jnp.int32(0), jnp.int32(0),
                     jnp.bool_(False)))
        thresh, cb, ceq = lax.cond(ok, lambda: (t1, cb1, ceq1), std_path)
        need = _K - cb

        def out_exact():
            @plsc.parallel_loop(0, _NV, unroll=8, carry=zeros_i)
            def _(i, run):
                off = gbase + i * _L
                v = data_v[pl.ds(off, _L)]
                r = jnp.maximum(v, 0.0)
                b = plsc.bitcast(r, jnp.int32)
                is_lt = b < thresh
                is_eq = b == thresh
                pref = plsc.cumsum(jnp.where(is_eq, 1, 0))
                zero_eq = jnp.logical_and(is_eq, (pref + run) <= need)
                z = jnp.logical_or(is_lt, zero_eq)
                data_v[pl.ds(off, _L)] = jnp.where(z, 0.0, r)
                return run + plsc.all_reduce_population_count(is_eq)

        def out_cheap():
            @plsc.parallel_loop(0, _NV, unroll=8)
            def _(i):
                off = gbase + i * _L
                v = data_v[pl.ds(off, _L)]
                r = jnp.maximum(v, 0.0)
                b = plsc.bitcast(r, jnp.int32)
                data_v[pl.ds(off, _L)] = jnp.where(b <= thresh, 0.0, r)

        lax.cond(ceq > need, out_exact, out_cheap)

        out_copy(g).start()
        return thresh

    lax.fori_loop(0, _GPW, per_group, jnp.int32(0))

    def drain(g, carry):
        out_copy(g).wait()
        return carry

    lax.fori_loop(0, _GPW, drain, 0)


_mesh = plsc.VectorSubcoreMesh(core_axis_name="c", subcore_axis_name="s",
                               num_cores=2, num_subcores=16)

_sc_call = pl.kernel(
    _sc_body,
    out_type=jax.ShapeDtypeStruct((_G * _N,), jnp.float32),
    mesh=_mesh,
    scratch_types=[
        pltpu.VMEM((_GPW * _N,), jnp.float32),
        pltpu.VMEM((_NBP,), jnp.int32),
        pltpu.SemaphoreType.DMA((_GPW,)),
        pltpu.SemaphoreType.DMA((_GPW,)),
    ],
    compiler_params=pltpu.CompilerParams(needs_layout_passes=False),
)


@jax.jit
def kernel(z_loga_expert):
    flat = z_loga_expert.reshape(_G * _N)
    out = _sc_call(flat)
    return out.reshape(_NL, _NE, _N)


# speedup vs baseline: 118.8437x; 1.5801x over previous
import jax
import jax.numpy as jnp
from jax import lax
from jax.experimental import pallas as pl
from jax.experimental.pallas import tpu as pltpu
from jax.experimental.pallas import tpu_sc as plsc

_NL, _NE, _N = 32, 8, 14336
_G = _NL * _NE
_K = _N // 2
_L = 16
_NW = 32
_GPW = _G // _NW
_NV = _N // _L
_NB = 2048
_NBP = 8192
_PH = _NBP // 2


def _take(x, idx):
    return x.at[idx].get(mode='promise_in_bounds')


def _scan_hist(hist_v, lo_v, cb_v, shift, nch):
    thr_v = _K - cb_v
    lanes = lax.iota(jnp.int32, _L)
    last = jnp.full((_L,), _L - 1, jnp.int32)

    tots = []
    for c in range(nch):
        acc = hist_v[pl.ds(c * 256, _L)]
        for i in range(1, 16):
            acc = acc + hist_v[pl.ds(c * 256 + i * _L, _L)]
        tots.append(_take(plsc.cumsum(acc), last))

    runb_v = jnp.zeros((_L,), jnp.int32)
    sel_v = jnp.zeros((_L,), jnp.int32)
    found_v = jnp.zeros((_L,), jnp.bool_)
    cum_c = jnp.zeros((_L,), jnp.int32)
    for c in range(nch):
        cum_n = cum_c + tots[c]
        hit = jnp.logical_and(jnp.logical_not(found_v), cum_n >= thr_v)
        sel_v = jnp.where(hit, c, sel_v)
        runb_v = jnp.where(hit, cum_c, runb_v)
        found_v = jnp.logical_or(found_v, hit)
        cum_c = cum_n

    jbase_s = jnp.max(sel_v) * 256
    jbase_v = sel_v * 256
    run_v = runb_v
    f2 = jnp.zeros((_L,), jnp.bool_)
    j_v = jnp.zeros((_L,), jnp.int32)
    cbadd_v = jnp.zeros((_L,), jnp.int32)
    ceq_v = jnp.zeros((_L,), jnp.int32)
    for i in range(16):
        h = hist_v[pl.ds(jbase_s + i * _L, _L)]
        cum = plsc.cumsum(h)
        m = (cum + run_v) >= thr_v
        lane = plsc.all_reduce_ffs(m)
        lane_c = jnp.minimum(lane, last)
        found_here = jnp.logical_and(lane < _L, jnp.logical_not(f2))
        exc_at = _take(cum - h, lane_c)
        h_at = _take(h, lane_c)
        j_v = jnp.where(found_here, jbase_v + i * _L + lane, j_v)
        cbadd_v = jnp.where(found_here, run_v + exc_at, cbadd_v)
        ceq_v = jnp.where(found_here, h_at, ceq_v)
        run_v = run_v + _take(cum, last)
        f2 = jnp.logical_or(f2, found_here)

    return (lo_v + jnp.left_shift(j_v, shift), cb_v + cbadd_v,
            found_v, ceq_v)


def _sc_body(x_hbm, out_hbm, data_v, hist_v, in_sem, out_sem):
    wid = lax.axis_index("s") * 2 + lax.axis_index("c")

    def in_copy(g):
        return pltpu.make_async_copy(
            x_hbm.at[wid, g],
            data_v.at[pl.ds(g * _N, _N)],
            in_sem.at[g])

    def out_copy(g):
        return pltpu.make_async_copy(
            data_v.at[pl.ds(g * _N, _N)],
            out_hbm.at[wid, g],
            out_sem.at[g])

    def fire(g, carry):
        in_copy(g).start()
        return carry

    lax.fori_loop(0, _GPW, fire, 0)

    ones_i = jnp.ones((_L,), jnp.int32)
    zeros_i = jnp.zeros((_L,), jnp.int32)

    def zero_hist(nvec):
        @plsc.parallel_loop(0, nvec, unroll=8)
        def _(i):
            hist_v[pl.ds(i * _L, _L)] = zeros_i

    def per_group(g, t_prev_v):
        gbase = g * _N
        in_copy(g).wait()

        def masked_round(lo_v, cb_v, shift, lo_s):
            zero_hist(_NB // _L)

            @plsc.parallel_loop(0, _NV, unroll=8)
            def _(i):
                v = data_v[pl.ds(gbase + i * _L, _L)]
                b = plsc.bitcast(jnp.maximum(v, 0.0), jnp.int32)
                d = b - lo_s
                idx = jnp.right_shift(d, shift)
                m = jnp.logical_and(d >= 0, idx < _NB)
                idx = jnp.clip(idx, 0, _NB - 1)
                plsc.addupdate_scatter(hist_v, [idx], ones_i, mask=m)

            return _scan_hist(hist_v, lo_v, cb_v, shift, _NB // 256)

        def pred_path():
            lo_p_v = t_prev_v - _PH
            zero_hist(_NBP // _L)

            @plsc.parallel_loop(0, _NV, unroll=8, carry=zeros_i)
            def below(i, cnt):
                v = data_v[pl.ds(gbase + i * _L, _L)]
                b = plsc.bitcast(jnp.maximum(v, 0.0), jnp.int32)
                d = b - lo_p_v
                m = jnp.logical_and(d >= 0, d < _NBP)
                idx = jnp.clip(d, 0, _NBP - 1)
                plsc.addupdate_scatter(hist_v, [idx], ones_i, mask=m)
                return cnt + jnp.where(d < 0, 1, 0)

            last = jnp.full((_L,), _L - 1, jnp.int32)
            h_below_v = _take(plsc.cumsum(below), last)
            t_v, cb_v, found_v, ceq_v = _scan_hist(
                hist_v, lo_p_v, h_below_v, 0, _NBP // 256)
            ok = jnp.any(jnp.logical_and(h_below_v < _K, found_v))
            return t_v, cb_v, ceq_v, ok

        def std_path():
            inf_v = jnp.full((_L,), jnp.inf, jnp.float32)
            zf_v = jnp.zeros((_L,), jnp.float32)

            @plsc.parallel_loop(0, _NV, unroll=8, carry=(inf_v, zf_v))
            def mm(i, c):
                mn, mx = c
                v = data_v[pl.ds(gbase + i * _L, _L)]
                r = jnp.maximum(v, 0.0)
                return jnp.minimum(mn, r), jnp.maximum(mx, r)

            mn_v, mx_v = mm
            mnb = jnp.min(plsc.bitcast(mn_v, jnp.int32))
            mxb = jnp.max(plsc.bitcast(mx_v, jnp.int32))
            rng1 = mxb - mnb
            s0 = jnp.int32(0)
            for t in range(11, 31):
                s0 = s0 + jnp.where(lax.shift_right_logical(rng1, t) != 0,
                                    1, 0)

            mnb_v = jnp.zeros((_L,), jnp.int32) + mnb
            lo_v, cb_v, _f, ceq_v = masked_round(
                mnb_v, jnp.zeros((_L,), jnp.int32), s0, mnb)
            s1 = jnp.maximum(s0 - 11, 0)

            def rb(lo_v=lo_v, cb_v=cb_v):
                return masked_round(lo_v, cb_v, s1, jnp.max(lo_v))

            lo_v, cb_v, _f, ceq_v = lax.cond(
                s0 > 0, rb,
                lambda: (lo_v, cb_v, jnp.ones((_L,), jnp.bool_), ceq_v))

            def rc(lo_v=lo_v, cb_v=cb_v):
                return masked_round(lo_v, cb_v, jnp.int32(0), jnp.max(lo_v))

            lo_v, cb_v, _f, ceq_v = lax.cond(
                s1 > 0, rc,
                lambda: (lo_v, cb_v, jnp.ones((_L,), jnp.bool_), ceq_v))
            return lo_v, cb_v, ceq_v

        zi = jnp.zeros((_L,), jnp.int32)
        t1_v, cb1_v, ceq1_v, ok = lax.cond(
            g > 0, pred_path, lambda: (zi, zi, zi, jnp.bool_(False)))
        thresh_v, cb_v, ceq_v = lax.cond(
            ok, lambda: (t1_v, cb1_v, ceq1_v), std_path)
        need_v = _K - cb_v

        def out_exact():
            @plsc.parallel_loop(0, _NV, unroll=8, carry=zeros_i)
            def _(i, run):
                off = gbase + i * _L
                v = data_v[pl.ds(off, _L)]
                r = jnp.maximum(v, 0.0)
                b = plsc.bitcast(r, jnp.int32)
                is_lt = b < thresh_v
                is_eq = b == thresh_v
                pref = plsc.cumsum(jnp.where(is_eq, 1, 0))
                zero_eq = jnp.logical_and(is_eq, (pref + run) <= need_v)
                z = jnp.logical_or(is_lt, zero_eq)
                data_v[pl.ds(off, _L)] = jnp.where(z, 0.0, r)
                return run + plsc.all_reduce_population_count(is_eq)

        def out_cheap():
            @plsc.parallel_loop(0, _NV, unroll=8)
            def _(i):
                off = gbase + i * _L
                v = data_v[pl.ds(off, _L)]
                r = jnp.maximum(v, 0.0)
                b = plsc.bitcast(r, jnp.int32)
                data_v[pl.ds(off, _L)] = jnp.where(b <= thresh_v, 0.0, r)

        lax.cond(jnp.any(ceq_v > need_v), out_exact, out_cheap)

        out_copy(g).start()
        return thresh_v

    lax.fori_loop(0, _GPW, per_group, jnp.zeros((_L,), jnp.int32))

    def drain(g, carry):
        out_copy(g).wait()
        return carry

    lax.fori_loop(0, _GPW, drain, 0)


_mesh = plsc.VectorSubcoreMesh(core_axis_name="c", subcore_axis_name="s",
                               num_cores=2, num_subcores=16)

_sc_call = pl.kernel(
    _sc_body,
    out_type=jax.ShapeDtypeStruct((_NL, _NE, _N), jnp.float32),
    mesh=_mesh,
    scratch_types=[
        pltpu.VMEM((_GPW * _N,), jnp.float32),
        pltpu.VMEM((_NBP,), jnp.int32),
        pltpu.SemaphoreType.DMA((_GPW,)),
        pltpu.SemaphoreType.DMA((_GPW,)),
    ],
    compiler_params=pltpu.CompilerParams(needs_layout_passes=False),
)


@jax.jit
def kernel(z_loga_expert):
    return _sc_call(z_loga_expert)


# speedup vs baseline: 127.0966x; 1.0694x over previous
import jax
import jax.numpy as jnp
from jax import lax
from jax.experimental import pallas as pl
from jax.experimental.pallas import tpu as pltpu
from jax.experimental.pallas import tpu_sc as plsc

_NL, _NE, _N = 32, 8, 14336
_G = _NL * _NE
_K = _N // 2
_L = 16
_NW = 32
_GPW = _G // _NW
_NV = _N // _L
_NB = 2048
_NBP = 8192
_PH = _NBP // 2


def _take(x, idx):
    return x.at[idx].get(mode='promise_in_bounds')


def _scan_hist(hist_v, lo_v, cb_v, shift, nch):
    thr_v = _K - cb_v
    lanes = lax.iota(jnp.int32, _L)
    last = jnp.full((_L,), _L - 1, jnp.int32)

    tots = []
    for c in range(nch):
        acc = hist_v[pl.ds(c * 256, _L)]
        for i in range(1, 16):
            acc = acc + hist_v[pl.ds(c * 256 + i * _L, _L)]
        tots.append(_take(plsc.cumsum(acc), last))

    runb_v = jnp.zeros((_L,), jnp.int32)
    sel_v = jnp.zeros((_L,), jnp.int32)
    found_v = jnp.zeros((_L,), jnp.bool_)
    cum_c = jnp.zeros((_L,), jnp.int32)
    for c in range(nch):
        cum_n = cum_c + tots[c]
        hit = jnp.logical_and(jnp.logical_not(found_v), cum_n >= thr_v)
        sel_v = jnp.where(hit, c, sel_v)
        runb_v = jnp.where(hit, cum_c, runb_v)
        found_v = jnp.logical_or(found_v, hit)
        cum_c = cum_n

    jbase_s = jnp.max(sel_v) * 256
    jbase_v = sel_v * 256
    run_v = runb_v
    f2 = jnp.zeros((_L,), jnp.bool_)
    j_v = jnp.zeros((_L,), jnp.int32)
    cbadd_v = jnp.zeros((_L,), jnp.int32)
    ceq_v = jnp.zeros((_L,), jnp.int32)
    for i in range(16):
        h = hist_v[pl.ds(jbase_s + i * _L, _L)]
        cum = plsc.cumsum(h)
        m = (cum + run_v) >= thr_v
        lane = plsc.all_reduce_ffs(m)
        lane_c = jnp.minimum(lane, last)
        found_here = jnp.logical_and(lane < _L, jnp.logical_not(f2))
        exc_at = _take(cum - h, lane_c)
        h_at = _take(h, lane_c)
        j_v = jnp.where(found_here, jbase_v + i * _L + lane, j_v)
        cbadd_v = jnp.where(found_here, run_v + exc_at, cbadd_v)
        ceq_v = jnp.where(found_here, h_at, ceq_v)
        run_v = run_v + _take(cum, last)
        f2 = jnp.logical_or(f2, found_here)

    return (lo_v + jnp.left_shift(j_v, shift), cb_v + cbadd_v,
            found_v, ceq_v)


def _sc_body(x_hbm, out_hbm, data_v, hist_v, in_sem, out_sem):
    wid = lax.axis_index("s") * 2 + lax.axis_index("c")

    def in_copy(g):
        return pltpu.make_async_copy(
            x_hbm.at[wid, g],
            data_v.at[pl.ds(g * _N, _N)],
            in_sem.at[g])

    def out_copy(g):
        return pltpu.make_async_copy(
            data_v.at[pl.ds(g * _N, _N)],
            out_hbm.at[wid, g],
            out_sem.at[g])

    def fire(g, carry):
        in_copy(g).start()
        return carry

    lax.fori_loop(0, _GPW, fire, 0)

    ones_i = jnp.ones((_L,), jnp.int32)
    zeros_i = jnp.zeros((_L,), jnp.int32)

    def zero_hist(nvec):
        @plsc.parallel_loop(0, nvec, unroll=8)
        def _(i):
            hist_v[pl.ds(i * _L, _L)] = zeros_i

    def per_group(g, t_prev_v):
        gbase = g * _N
        in_copy(g).wait()

        def masked_round(lo_v, cb_v, shift, lo_s):
            zero_hist(_NB // _L)

            @plsc.parallel_loop(0, _NV, unroll=8)
            def _(i):
                v = data_v[pl.ds(gbase + i * _L, _L)]
                b = plsc.bitcast(jnp.maximum(v, 0.0), jnp.int32)
                idx = jnp.right_shift(b - lo_s, shift)
                m = plsc.bitcast(idx, jnp.uint32) < jnp.uint32(_NB)
                plsc.addupdate_scatter(hist_v, [idx], ones_i, mask=m)

            return _scan_hist(hist_v, lo_v, cb_v, shift, _NB // 256)

        def pred_path():
            lo_p_v = t_prev_v - _PH
            zero_hist(_NBP // _L)

            @plsc.parallel_loop(0, _NV, unroll=8, carry=zeros_i)
            def below(i, cnt):
                v = data_v[pl.ds(gbase + i * _L, _L)]
                b = plsc.bitcast(jnp.maximum(v, 0.0), jnp.int32)
                d = b - lo_p_v
                m = plsc.bitcast(d, jnp.uint32) < jnp.uint32(_NBP)
                plsc.addupdate_scatter(hist_v, [d], ones_i, mask=m)
                return cnt + jnp.where(d < 0, 1, 0)

            last = jnp.full((_L,), _L - 1, jnp.int32)
            h_below_v = _take(plsc.cumsum(below), last)
            t_v, cb_v, found_v, ceq_v = _scan_hist(
                hist_v, lo_p_v, h_below_v, 0, _NBP // 256)
            ok = jnp.any(jnp.logical_and(h_below_v < _K, found_v))
            return t_v, cb_v, ceq_v, ok

        def std_path():
            inf_v = jnp.full((_L,), jnp.inf, jnp.float32)
            zf_v = jnp.zeros((_L,), jnp.float32)

            @plsc.parallel_loop(0, _NV, unroll=8, carry=(inf_v, zf_v))
            def mm(i, c):
                mn, mx = c
                v = data_v[pl.ds(gbase + i * _L, _L)]
                r = jnp.maximum(v, 0.0)
                return jnp.minimum(mn, r), jnp.maximum(mx, r)

            mn_v, mx_v = mm
            mnb = jnp.min(plsc.bitcast(mn_v, jnp.int32))
            mxb = jnp.max(plsc.bitcast(mx_v, jnp.int32))
            rng1 = mxb - mnb
            s0 = jnp.int32(0)
            for t in range(11, 31):
                s0 = s0 + jnp.where(lax.shift_right_logical(rng1, t) != 0,
                                    1, 0)

            mnb_v = jnp.zeros((_L,), jnp.int32) + mnb
            lo_v, cb_v, _f, ceq_v = masked_round(
                mnb_v, jnp.zeros((_L,), jnp.int32), s0, mnb)
            s1 = jnp.maximum(s0 - 11, 0)

            def rb(lo_v=lo_v, cb_v=cb_v):
                return masked_round(lo_v, cb_v, s1, jnp.max(lo_v))

            lo_v, cb_v, _f, ceq_v = lax.cond(
                s0 > 0, rb,
                lambda: (lo_v, cb_v, jnp.ones((_L,), jnp.bool_), ceq_v))

            def rc(lo_v=lo_v, cb_v=cb_v):
                return masked_round(lo_v, cb_v, jnp.int32(0), jnp.max(lo_v))

            lo_v, cb_v, _f, ceq_v = lax.cond(
                s1 > 0, rc,
                lambda: (lo_v, cb_v, jnp.ones((_L,), jnp.bool_), ceq_v))
            return lo_v, cb_v, ceq_v

        zi = jnp.zeros((_L,), jnp.int32)
        t1_v, cb1_v, ceq1_v, ok = lax.cond(
            g > 0, pred_path, lambda: (zi, zi, zi, jnp.bool_(False)))
        thresh_v, cb_v, ceq_v = lax.cond(
            ok, lambda: (t1_v, cb1_v, ceq1_v), std_path)
        need_v = _K - cb_v

        def out_exact():
            @plsc.parallel_loop(0, _NV, unroll=8, carry=zeros_i)
            def _(i, run):
                off = gbase + i * _L
                v = data_v[pl.ds(off, _L)]
                r = jnp.maximum(v, 0.0)
                b = plsc.bitcast(r, jnp.int32)
                is_lt = b < thresh_v
                is_eq = b == thresh_v
                pref = plsc.cumsum(jnp.where(is_eq, 1, 0))
                zero_eq = jnp.logical_and(is_eq, (pref + run) <= need_v)
                z = jnp.logical_or(is_lt, zero_eq)
                data_v[pl.ds(off, _L)] = jnp.where(z, 0.0, r)
                return run + plsc.all_reduce_population_count(is_eq)

        def out_cheap():
            @plsc.parallel_loop(0, _NV, unroll=8)
            def _(i):
                off = gbase + i * _L
                v = data_v[pl.ds(off, _L)]
                r = jnp.maximum(v, 0.0)
                b = plsc.bitcast(r, jnp.int32)
                data_v[pl.ds(off, _L)] = jnp.where(b <= thresh_v, 0.0, r)

        lax.cond(jnp.any(ceq_v > need_v), out_exact, out_cheap)

        out_copy(g).start()
        return thresh_v

    lax.fori_loop(0, _GPW, per_group, jnp.zeros((_L,), jnp.int32))

    def drain(g, carry):
        out_copy(g).wait()
        return carry

    lax.fori_loop(0, _GPW, drain, 0)


_mesh = plsc.VectorSubcoreMesh(core_axis_name="c", subcore_axis_name="s",
                               num_cores=2, num_subcores=16)

_sc_call = pl.kernel(
    _sc_body,
    out_type=jax.ShapeDtypeStruct((_NL, _NE, _N), jnp.float32),
    mesh=_mesh,
    scratch_types=[
        pltpu.VMEM((_GPW * _N,), jnp.float32),
        pltpu.VMEM((_NBP,), jnp.int32),
        pltpu.SemaphoreType.DMA((_GPW,)),
        pltpu.SemaphoreType.DMA((_GPW,)),
    ],
    compiler_params=pltpu.CompilerParams(needs_layout_passes=False),
)


@jax.jit
def kernel(z_loga_expert):
    return _sc_call(z_loga_expert)
